# Initial kernel scaffold; baseline (speedup 1.0000x reference)
import jax
import jax.numpy as jnp
from jax.experimental import pallas as pl


def kernel(x, features_2d):
    raise NotImplementedError("write your pallas kernel here")


# speedup vs baseline: 11.5224x; 11.5224x over previous
import functools

import jax
import jax.numpy as jnp
from jax import lax
from jax.experimental import pallas as pl
from jax.experimental.pallas import tpu as pltpu
from jax.experimental.pallas import tpu_sc as plsc

GRID = 512
C = 48
NPLANES = 3
NCORES = 2
NSUB = 16
NW = NCORES * NSUB
K = 64
PLANE_AXES = ((1, 2), (0, 1), (2, 0))


def _worker_body(n_points, x_hbm, table_hbm, out_hbm, xv, idxv, wref, rows, outv, gsem):
    wid = lax.axis_index("c") * NSUB + lax.axis_index("s")
    pw = n_points // NW
    nchunks = pw // K

    def chunk_body(t, _):
        base = wid * pw + t * K
        for p in range(3):
            pltpu.sync_copy(x_hbm.at[p, pl.ds(base, K)], xv.at[p])

        for g in range(K // 16):
            sl = pl.ds(g * 16, 16)
            xc = [xv[p, sl] for p in range(3)]
            for p, (a, b) in enumerate(PLANE_AXES):
                u = xc[a] * 128.0 + 256.0
                v = xc[b] * 128.0 + 256.0
                iu = u.astype(jnp.int32)
                iv = v.astype(jnp.int32)
                wu1 = u - iu.astype(jnp.float32)
                wv1 = v - iv.astype(jnp.float32)
                wu0 = 1.0 - wu1
                wv0 = (1.0 - wv1) * (1.0 / 3.0)
                wv1 = wv1 * (1.0 / 3.0)
                r00 = iv * GRID + iu + p * (GRID * GRID)
                idxv[4 * p + 0, sl] = r00
                idxv[4 * p + 1, sl] = r00 + 1
                idxv[4 * p + 2, sl] = r00 + GRID
                idxv[4 * p + 3, sl] = r00 + GRID + 1
                wref[4 * p + 0, sl] = wu0 * wv0
                wref[4 * p + 1, sl] = wu1 * wv0
                wref[4 * p + 2, sl] = wu0 * wv1
                wref[4 * p + 3, sl] = wu1 * wv1

        copies = [
            pltpu.async_copy(table_hbm.at[idxv.at[r]], rows.at[r], gsem)
            for r in range(12)
        ]
        for cp in copies:
            cp.wait()

        def acc_body(g2, _):
            sl = pl.ds(g2 * 16, 16)
            wv = [wref[r, sl] for r in range(12)]
            for j in range(16):
                i = g2 * 16 + j
                acc = [jnp.zeros((16,), jnp.float32) for _ in range(3)]
                for r in range(12):
                    w = wv[r][j]
                    for cblk in range(3):
                        acc[cblk] = acc[cblk] + w * rows[r, i, pl.ds(cblk * 16, 16)]
                for cblk in range(3):
                    outv[i, pl.ds(cblk * 16, 16)] = acc[cblk]
            return 0

        lax.fori_loop(0, K // 16, acc_body, 0)

        pltpu.sync_copy(outv, out_hbm.at[pl.ds(base, K)])
        return 0

    lax.fori_loop(0, nchunks, chunk_body, 0)


@functools.partial(jax.jit, static_argnames=("n_points",))
def _triplane_sc(xt, table, n_points):
    mesh = plsc.VectorSubcoreMesh(
        core_axis_name="c", subcore_axis_name="s",
        num_cores=NCORES, num_subcores=NSUB,
    )
    body = functools.partial(_worker_body, n_points)
    return pl.kernel(
        body,
        out_type=jax.ShapeDtypeStruct((n_points, C), jnp.float32),
        mesh=mesh,
        scratch_types=[
            pltpu.VMEM((3, K), jnp.float32),
            pltpu.VMEM((12, K), jnp.int32),
            pltpu.VMEM((12, K), jnp.float32),
            pltpu.VMEM((12, K, C), jnp.float32),
            pltpu.VMEM((K, C), jnp.float32),
            pltpu.SemaphoreType.DMA,
        ],
        compiler_params=pltpu.CompilerParams(use_tc_tiling_on_sc=False),
    )(xt, table)


def kernel(x, features_2d):
    n = x.shape[0]
    xt = lax.reduce_precision(x.T, exponent_bits=8, mantissa_bits=7)
    table = features_2d.reshape(NPLANES * GRID * GRID, C)
    return _triplane_sc(xt, table, n)


# speedup vs baseline: 16.4421x; 1.4270x over previous
import functools

import jax
import jax.numpy as jnp
from jax import lax
from jax.experimental import pallas as pl
from jax.experimental.pallas import tpu as pltpu
from jax.experimental.pallas import tpu_sc as plsc

GRID = 512
C = 48
NPLANES = 3
NCORES = 2
NSUB = 16
NW = NCORES * NSUB
K = 64
XSUP = 8
PLANE_AXES = ((1, 2), (0, 1), (2, 0))


def _worker_body(n_points, x_hbm, table_hbm, out_hbm,
                 xv, idxv0, wref0, rows0, outv0, idxv1, wref1, rows1, outv1,
                 gsem0, gsem1, osem0, osem1):
    wid = lax.axis_index("c") * NSUB + lax.axis_index("s")
    pw = n_points // NW
    nchunks = pw // K
    wbase = wid * pw
    bufs = ((idxv0, wref0, rows0, outv0, gsem0, osem0),
            (idxv1, wref1, rows1, outv1, gsem1, osem1))

    def stage_x(sup_base):
        for p in range(3):
            pltpu.sync_copy(x_hbm.at[p, pl.ds(sup_base, XSUP * K)], xv.at[p])

    def compute_and_fire(t, idxv, wref, rows, gsem):
        col0 = (t % XSUP) * K
        for g in range(K // 16):
            slx = pl.ds(col0 + g * 16, 16)
            sl = pl.ds(g * 16, 16)
            xc = [xv[p, slx] for p in range(3)]
            for p, (a, b) in enumerate(PLANE_AXES):
                u = xc[a] * 128.0 + 256.0
                v = xc[b] * 128.0 + 256.0
                iu = u.astype(jnp.int32)
                iv = v.astype(jnp.int32)
                wu1 = u - iu.astype(jnp.float32)
                wv1 = v - iv.astype(jnp.float32)
                wu0 = 1.0 - wu1
                wv0 = (1.0 - wv1) * (1.0 / 3.0)
                wv1 = wv1 * (1.0 / 3.0)
                r00 = iv * GRID + iu + p * (GRID * GRID)
                idxv[4 * p + 0, sl] = r00
                idxv[4 * p + 1, sl] = r00 + 1
                idxv[4 * p + 2, sl] = r00 + GRID
                idxv[4 * p + 3, sl] = r00 + GRID + 1
                wref[4 * p + 0, sl] = wu0 * wv0
                wref[4 * p + 1, sl] = wu1 * wv0
                wref[4 * p + 2, sl] = wu0 * wv1
                wref[4 * p + 3, sl] = wu1 * wv1
        for r in range(12):
            pltpu.async_copy(table_hbm.at[idxv.at[r]], rows.at[r], gsem)

    def wait_gathers(rows, gsem):
        for r in range(12):
            pltpu.make_async_copy(table_hbm.at[pl.ds(0, K)], rows.at[r], gsem).wait()

    def accumulate(rows, wref, outv):
        def acc_body(g2, _):
            sl = pl.ds(g2 * 16, 16)
            wv = [wref[r, sl] for r in range(12)]
            for j in range(16):
                i = g2 * 16 + j
                acc = [jnp.zeros((16,), jnp.float32) for _ in range(3)]
                for r in range(12):
                    w = wv[r][j]
                    for cblk in range(3):
                        acc[cblk] = acc[cblk] + w * rows[r, i, pl.ds(cblk * 16, 16)]
                for cblk in range(3):
                    outv[i, pl.ds(cblk * 16, 16)] = acc[cblk]
            return 0

        lax.fori_loop(0, K // 16, acc_body, 0)

    stage_x(wbase)
    compute_and_fire(0, bufs[0][0], bufs[0][1], bufs[0][2], bufs[0][4])

    def chunk_body(t2, _):
        for phase in range(2):
            t = t2 * 2 + phase
            idxv, wref, rows, outv, gsem, osem = bufs[phase]
            nidxv, nwref, nrows, _, ngsem, _ = bufs[1 - phase]
            tn = t + 1

            @pl.when(tn < nchunks)
            def _():
                @pl.when(tn % XSUP == 0)
                def _():
                    stage_x(wbase + tn * K)
                compute_and_fire(tn, nidxv, nwref, nrows, ngsem)

            wait_gathers(rows, gsem)

            @pl.when(t >= 2)
            def _():
                pltpu.make_async_copy(outv, out_hbm.at[pl.ds(0, K)], osem).wait()

            accumulate(rows, wref, outv)
            pltpu.async_copy(outv, out_hbm.at[pl.ds(wbase + t * K, K)], osem)
        return 0

    lax.fori_loop(0, nchunks // 2, chunk_body, 0)

    for phase in range(2):
        _, _, _, outv, _, osem = bufs[phase]
        pltpu.make_async_copy(outv, out_hbm.at[pl.ds(0, K)], osem).wait()


@functools.partial(jax.jit, static_argnames=("n_points",))
def _triplane_sc(xt, table, n_points):
    mesh = plsc.VectorSubcoreMesh(
        core_axis_name="c", subcore_axis_name="s",
        num_cores=NCORES, num_subcores=NSUB,
    )
    body = functools.partial(_worker_body, n_points)
    buf = [
        pltpu.VMEM((12, K), jnp.int32),
        pltpu.VMEM((12, K), jnp.float32),
        pltpu.VMEM((12, K, C), jnp.float32),
        pltpu.VMEM((K, C), jnp.float32),
    ]
    return pl.kernel(
        body,
        out_type=jax.ShapeDtypeStruct((n_points, C), jnp.float32),
        mesh=mesh,
        scratch_types=(
            [pltpu.VMEM((3, XSUP * K), jnp.float32)]
            + buf + buf
            + [pltpu.SemaphoreType.DMA] * 4
        ),
        compiler_params=pltpu.CompilerParams(use_tc_tiling_on_sc=False),
    )(xt, table)


def kernel(x, features_2d):
    n = x.shape[0]
    xt = lax.reduce_precision(x.T, exponent_bits=8, mantissa_bits=7)
    table = features_2d.reshape(NPLANES * GRID * GRID, C)
    return _triplane_sc(xt, table, n)


# speedup vs baseline: 17.2537x; 1.0494x over previous
import functools

import jax
import jax.numpy as jnp
from jax import lax
from jax.experimental import pallas as pl
from jax.experimental.pallas import tpu as pltpu
from jax.experimental.pallas import tpu_sc as plsc

GRID = 512
C = 48
NPLANES = 3
NCORES = 2
NSUB = 16
NW = NCORES * NSUB
K = 64
XSUP = 8
PLANE_AXES = ((1, 2), (0, 1), (2, 0))


def _worker_body(n_points, x_hbm, table_hbm, out_hbm,
                 xv, idxv0, wref0, rows0, outv0, idxv1, wref1, rows1, outv1,
                 gsem0, gsem1, osem0, osem1):
    wid = lax.axis_index("c") * NSUB + lax.axis_index("s")
    pw = n_points // NW
    nchunks = pw // K
    wbase = wid * pw
    bufs = ((idxv0, wref0, rows0, outv0, gsem0, osem0),
            (idxv1, wref1, rows1, outv1, gsem1, osem1))

    def stage_x(sup_base):
        for p in range(3):
            pltpu.sync_copy(x_hbm.at[p, pl.ds(sup_base, XSUP * K)], xv.at[p])

    def compute_and_fire(t, idxv, wref, rows, gsem):
        col0 = (t % XSUP) * K
        for g in range(K // 16):
            slx = pl.ds(col0 + g * 16, 16)
            sl = pl.ds(g * 16, 16)
            xc = [xv[p, slx] for p in range(3)]
            for p, (a, b) in enumerate(PLANE_AXES):
                u = xc[a] * 128.0 + 256.0
                v = xc[b] * 128.0 + 256.0
                iu = u.astype(jnp.int32)
                iv = v.astype(jnp.int32)
                wu1 = u - iu.astype(jnp.float32)
                wv1 = v - iv.astype(jnp.float32)
                wu0 = 1.0 - wu1
                wv0 = (1.0 - wv1) * (1.0 / 3.0)
                wv1 = wv1 * (1.0 / 3.0)
                r00 = iv * GRID + iu + p * (GRID * GRID)
                for q, off in enumerate((0, 1, GRID, GRID + 1)):
                    r = 4 * p + q
                    idxv[r // 2, pl.ds((r % 2) * K + g * 16, 16)] = r00 + off
                wref[4 * p + 0, sl] = wu0 * wv0
                wref[4 * p + 1, sl] = wu1 * wv0
                wref[4 * p + 2, sl] = wu0 * wv1
                wref[4 * p + 3, sl] = wu1 * wv1
        for r2 in range(6):
            pltpu.async_copy(table_hbm.at[idxv.at[r2]], rows.at[r2], gsem)

    def wait_gathers(idxv, rows, gsem):
        for r2 in range(6):
            pltpu.make_async_copy(table_hbm.at[idxv.at[r2]], rows.at[r2], gsem).wait()

    def accumulate(rows, wref, outv):
        def acc_body(g2, _):
            sl = pl.ds(g2 * 16, 16)
            wv = [wref[r, sl] for r in range(12)]
            for j in range(16):
                i = g2 * 16 + j
                ws = [wv[r][j] for r in range(12)]
                for cblk in range(3):
                    csl = pl.ds(cblk * 16, 16)
                    parts = [
                        ws[2 * q] * rows[q, i, csl]
                        + ws[2 * q + 1] * rows[q, K + i, csl]
                        for q in range(6)
                    ]
                    s0 = parts[0] + parts[1]
                    s1 = parts[2] + parts[3]
                    s2 = parts[4] + parts[5]
                    outv[i, csl] = s0 + s1 + s2
            return 0

        lax.fori_loop(0, K // 16, acc_body, 0)

    stage_x(wbase)
    compute_and_fire(0, bufs[0][0], bufs[0][1], bufs[0][2], bufs[0][4])

    def chunk_body(t2, _):
        for phase in range(2):
            t = t2 * 2 + phase
            idxv, wref, rows, outv, gsem, osem = bufs[phase]
            nidxv, nwref, nrows, _, ngsem, _ = bufs[1 - phase]
            tn = t + 1

            @pl.when(tn < nchunks)
            def _():
                @pl.when(tn % XSUP == 0)
                def _():
                    stage_x(wbase + tn * K)
                compute_and_fire(tn, nidxv, nwref, nrows, ngsem)

            wait_gathers(idxv, rows, gsem)

            @pl.when(t >= 2)
            def _():
                pltpu.make_async_copy(outv, out_hbm.at[pl.ds(0, K)], osem).wait()

            accumulate(rows, wref, outv)
            pltpu.async_copy(outv, out_hbm.at[pl.ds(wbase + t * K, K)], osem)
        return 0

    lax.fori_loop(0, nchunks // 2, chunk_body, 0)

    for phase in range(2):
        _, _, _, outv, _, osem = bufs[phase]
        pltpu.make_async_copy(outv, out_hbm.at[pl.ds(0, K)], osem).wait()


@functools.partial(jax.jit, static_argnames=("n_points",))
def _triplane_sc(xt, table, n_points):
    mesh = plsc.VectorSubcoreMesh(
        core_axis_name="c", subcore_axis_name="s",
        num_cores=NCORES, num_subcores=NSUB,
    )
    body = functools.partial(_worker_body, n_points)
    buf = [
        pltpu.VMEM((6, 2 * K), jnp.int32),
        pltpu.VMEM((12, K), jnp.float32),
        pltpu.VMEM((6, 2 * K, C), jnp.float32),
        pltpu.VMEM((K, C), jnp.float32),
    ]
    return pl.kernel(
        body,
        out_type=jax.ShapeDtypeStruct((n_points, C), jnp.float32),
        mesh=mesh,
        scratch_types=(
            [pltpu.VMEM((3, XSUP * K), jnp.float32)]
            + buf + buf
            + [pltpu.SemaphoreType.DMA] * 4
        ),
        compiler_params=pltpu.CompilerParams(use_tc_tiling_on_sc=False),
    )(xt, table)


def kernel(x, features_2d):
    n = x.shape[0]
    xt = lax.reduce_precision(x.T, exponent_bits=8, mantissa_bits=7)
    table = features_2d.reshape(NPLANES * GRID * GRID, C)
    return _triplane_sc(xt, table, n)
